# Initial kernel scaffold; baseline (speedup 1.0000x reference)
import jax
import jax.numpy as jnp
from jax.experimental import pallas as pl


def kernel(x, pos, idx, W_self, W_edge):
    raise NotImplementedError("write your pallas kernel here")


# speedup vs baseline: 16.5397x; 16.5397x over previous
import functools

import jax
import jax.numpy as jnp
from jax import lax
from jax.experimental import pallas as pl
from jax.experimental.pallas import tpu as pltpu
from jax.experimental.pallas import tpu_sc as plsc

_B, _N, _C, _K = 4, 10000, 128, 16
_R = _B * _N
_D = 144
_NC, _NS = 2, 16
_NW = _NC * _NS
_CH = 8
_RP = 40960
_PER_W = _RP // _NW
_NCHUNK = _PER_W // _CH

@functools.cache
def _make_sc_gather_mean():
    mesh = plsc.VectorSubcoreMesh(
        core_axis_name="c", subcore_axis_name="s",
        num_cores=_NC, num_subcores=_NS)

    @functools.partial(
        pl.kernel,
        out_type=(
            jax.ShapeDtypeStruct((_RP, _C), jnp.float32),
            jax.ShapeDtypeStruct((_RP, 16), jnp.float32),
        ),
        mesh=mesh,
        compiler_params=pltpu.CompilerParams(use_tc_tiling_on_sc=False),
        scratch_types=[
            pltpu.VMEM((_CH * _K,), jnp.int32),
            pltpu.VMEM((_CH * _K, _D), jnp.float32),
            pltpu.VMEM((_CH, _C), jnp.float32),
            pltpu.VMEM((_CH, 16), jnp.float32),
            pltpu.SemaphoreType.DMA,
        ],
    )
    def _sc_gather_mean(tab_hbm, idxg_hbm, outx_hbm, outm_hbm,
                        idx_v, rows_v, accx_v, accm_v, sem):
        w = lax.axis_index("s") * _NC + lax.axis_index("c")
        base = w * _PER_W

        @pl.loop(0, _NCHUNK)
        def _chunk(j):
            row0 = base + j * _CH
            pltpu.sync_copy(idxg_hbm.at[pl.ds(row0 * _K, _CH * _K)], idx_v)
            pltpu.async_copy(tab_hbm.at[idx_v], rows_v, sem).wait()
            inv_k = 1.0 / _K
            for c in range(_CH):
                r0 = c * _K
                for d in range(_D // 16):
                    acc = rows_v[r0, pl.ds(d * 16, 16)]
                    for k in range(1, _K):
                        acc = acc + rows_v[r0 + k, pl.ds(d * 16, 16)]
                    acc = acc * inv_k
                    if d < _C // 16:
                        accx_v[c, pl.ds(d * 16, 16)] = acc
                    else:
                        accm_v[c, :] = acc
            pltpu.sync_copy(accx_v, outx_hbm.at[pl.ds(row0, _CH)])
            pltpu.sync_copy(accm_v, outm_hbm.at[pl.ds(row0, _CH)])

    return _sc_gather_mean


def _tc_body(x_ref, gx_ref, gm_ref, pq_ref, wst_ref, wext_ref, sm_ref, o_ref):
    xr = x_ref[...]
    acc = jnp.dot(xr, wst_ref[...], preferred_element_type=jnp.float32)
    acc = acc + jnp.dot(gx_ref[...], wext_ref[...],
                        preferred_element_type=jnp.float32)
    gm = gm_ref[...]
    pq = pq_ref[...]
    pos3 = pq[:, 0:3]
    q = pq[:, 3:4]
    mp = gm[:, 0:3]
    mq = gm[:, 3:4]
    rel = pos3 - mp
    dterm = q - 2.0 * jnp.sum(pos3 * mp, axis=1, keepdims=True) + mq
    sm = sm_ref[...]
    acc = acc + (rel[:, 0:1] * sm[0:1, :] + rel[:, 1:2] * sm[1:2, :]
                 + rel[:, 2:3] * sm[2:3, :] + dterm * sm[3:4, :])
    o_ref[...] = jnp.where(acc >= 0, acc, 0.2 * acc)


_BR = 1000

_tc_dense = pl.pallas_call(
    _tc_body,
    grid=(_R // _BR,),
    in_specs=[
        pl.BlockSpec((_BR, _C), lambda i: (i, 0)),
        pl.BlockSpec((_BR, _C), lambda i: (i, 0)),
        pl.BlockSpec((_BR, 16), lambda i: (i, 0)),
        pl.BlockSpec((_BR, 8), lambda i: (i, 0)),
        pl.BlockSpec((_C, _C), lambda i: (0, 0)),
        pl.BlockSpec((_C, _C), lambda i: (0, 0)),
        pl.BlockSpec((8, _C), lambda i: (0, 0)),
    ],
    out_specs=pl.BlockSpec((_BR, _C), lambda i: (i, 0)),
    out_shape=jax.ShapeDtypeStruct((_R, _C), jnp.float32),
)


def kernel(x, pos, idx, W_self, W_edge):
    B, N, C = x.shape
    q = jnp.sum(pos * pos, axis=-1, keepdims=True)
    pad = jnp.zeros((B, N, _D - C - 4), jnp.float32)
    tab = jnp.concatenate([x, pos, q, pad], axis=-1).reshape(_R, _D)
    idxg = (idx + (jnp.arange(B, dtype=jnp.int32) * N)[:, None, None]
            ).reshape(_R * _K)
    idxg = jnp.pad(idxg, (0, (_RP - _R) * _K))
    gx, gm = _make_sc_gather_mean()(tab, idxg)

    xf = x.reshape(_R, C)
    pq = jnp.concatenate(
        [pos, q, jnp.zeros((B, N, 4), jnp.float32)], axis=-1).reshape(_R, 8)
    wst = W_self.T
    wext = W_edge[:, :C].T
    sm = jnp.zeros((8, C), jnp.float32)
    sm = sm.at[0:3, :].set(W_edge[:, C:C + 3].T)
    sm = sm.at[3, :].set(W_edge[:, C + 3])
    out = _tc_dense(xf, gx, gm, pq, wst, wext, sm)
    return out.reshape(B, N, C)


# speedup vs baseline: 21.6991x; 1.3119x over previous
import functools

import jax
import jax.numpy as jnp
from jax import lax
from jax.experimental import pallas as pl
from jax.experimental.pallas import tpu as pltpu
from jax.experimental.pallas import tpu_sc as plsc

_B, _N, _C, _K = 4, 10000, 128, 16
_R = _B * _N
_D = 144
_NC, _NS = 2, 16
_NW = _NC * _NS
_CH = 8
_RP = 40960
_PER_W = _RP // _NW
_NCHUNK = _PER_W // _CH

@functools.cache
def _make_sc_gather_mean():
    mesh = plsc.VectorSubcoreMesh(
        core_axis_name="c", subcore_axis_name="s",
        num_cores=_NC, num_subcores=_NS)

    @functools.partial(
        pl.kernel,
        out_type=(
            jax.ShapeDtypeStruct((_RP, _C), jnp.float32),
            jax.ShapeDtypeStruct((_RP, 16), jnp.float32),
        ),
        mesh=mesh,
        compiler_params=pltpu.CompilerParams(use_tc_tiling_on_sc=False),
        scratch_types=[
            pltpu.VMEM((_PER_W * _K,), jnp.int32),
            pltpu.VMEM((2, _CH * _K, _D), jnp.float32),
            pltpu.VMEM((2, _CH, _C), jnp.float32),
            pltpu.VMEM((2, _CH, 16), jnp.float32),
            pltpu.SemaphoreType.DMA((2,)),
            pltpu.SemaphoreType.DMA((2,)),
            pltpu.SemaphoreType.DMA((2,)),
        ],
    )
    def _sc_gather_mean(tab_hbm, idxg_hbm, outx_hbm, outm_hbm,
                        idx_v, rows_v, accx_v, accm_v, rsem, sxsem, smsem):
        w = lax.axis_index("s") * _NC + lax.axis_index("c")
        base = w * _PER_W

        pltpu.sync_copy(idxg_hbm.at[pl.ds(base * _K, _PER_W * _K)], idx_v)

        def _gather(j, s):
            pltpu.async_copy(
                tab_hbm.at[idx_v.at[pl.ds(j * _CH * _K, _CH * _K)]],
                rows_v.at[s], rsem.at[s])

        def _gather_wait(s):
            pltpu.make_async_copy(
                tab_hbm.at[idx_v.at[pl.ds(0, _CH * _K)]],
                rows_v.at[s], rsem.at[s]).wait()

        def _store(j, s):
            row0 = base + j * _CH
            pltpu.async_copy(accx_v.at[s], outx_hbm.at[pl.ds(row0, _CH)],
                             sxsem.at[s])
            pltpu.async_copy(accm_v.at[s], outm_hbm.at[pl.ds(row0, _CH)],
                             smsem.at[s])

        def _store_wait(s):
            pltpu.make_async_copy(accx_v.at[s], outx_hbm.at[pl.ds(0, _CH)],
                                  sxsem.at[s]).wait()
            pltpu.make_async_copy(accm_v.at[s], outm_hbm.at[pl.ds(0, _CH)],
                                  smsem.at[s]).wait()

        _gather(0, 0)
        inv_k = 1.0 / _K

        @pl.loop(0, _NCHUNK, step=2)
        def _pair(j0):
            for s in (0, 1):
                j = j0 + s

                @pl.when(j + 1 < _NCHUNK)
                def _():
                    _gather(j + 1, 1 - s)

                _gather_wait(s)

                @pl.when(j >= 2)
                def _():
                    _store_wait(s)

                for c in range(_CH):
                    r0 = c * _K
                    for d in range(_D // 16):
                        acc = rows_v[s, r0, pl.ds(d * 16, 16)]
                        for k in range(1, _K):
                            acc = acc + rows_v[s, r0 + k, pl.ds(d * 16, 16)]
                        acc = acc * inv_k
                        if d < _C // 16:
                            accx_v[s, c, pl.ds(d * 16, 16)] = acc
                        else:
                            accm_v[s, c, :] = acc

                _store(j, s)

        _store_wait(0)
        _store_wait(1)

    return _sc_gather_mean


def _tc_body(x_ref, gx_ref, gm_ref, pq_ref, wst_ref, wext_ref, sm_ref, o_ref):
    xr = x_ref[...]
    acc = jnp.dot(xr, wst_ref[...], preferred_element_type=jnp.float32)
    acc = acc + jnp.dot(gx_ref[...], wext_ref[...],
                        preferred_element_type=jnp.float32)
    gm = gm_ref[...]
    pq = pq_ref[...]
    pos3 = pq[:, 0:3]
    q = pq[:, 3:4]
    mp = gm[:, 0:3]
    mq = gm[:, 3:4]
    rel = pos3 - mp
    dterm = q - 2.0 * jnp.sum(pos3 * mp, axis=1, keepdims=True) + mq
    sm = sm_ref[...]
    acc = acc + (rel[:, 0:1] * sm[0:1, :] + rel[:, 1:2] * sm[1:2, :]
                 + rel[:, 2:3] * sm[2:3, :] + dterm * sm[3:4, :])
    o_ref[...] = jnp.where(acc >= 0, acc, 0.2 * acc)


_BR = 1000

_tc_dense = pl.pallas_call(
    _tc_body,
    grid=(_R // _BR,),
    in_specs=[
        pl.BlockSpec((_BR, _C), lambda i: (i, 0)),
        pl.BlockSpec((_BR, _C), lambda i: (i, 0)),
        pl.BlockSpec((_BR, 16), lambda i: (i, 0)),
        pl.BlockSpec((_BR, 8), lambda i: (i, 0)),
        pl.BlockSpec((_C, _C), lambda i: (0, 0)),
        pl.BlockSpec((_C, _C), lambda i: (0, 0)),
        pl.BlockSpec((8, _C), lambda i: (0, 0)),
    ],
    out_specs=pl.BlockSpec((_BR, _C), lambda i: (i, 0)),
    out_shape=jax.ShapeDtypeStruct((_R, _C), jnp.float32),
)


def kernel(x, pos, idx, W_self, W_edge):
    B, N, C = x.shape
    q = jnp.sum(pos * pos, axis=-1, keepdims=True)
    pad = jnp.zeros((B, N, _D - C - 4), jnp.float32)
    tab = jnp.concatenate([x, pos, q, pad], axis=-1).reshape(_R, _D)
    idxg = (idx + (jnp.arange(B, dtype=jnp.int32) * N)[:, None, None]
            ).reshape(_R * _K)
    idxg = jnp.pad(idxg, (0, (_RP - _R) * _K))
    gx, gm = _make_sc_gather_mean()(tab, idxg)

    xf = x.reshape(_R, C)
    pq = jnp.concatenate(
        [pos, q, jnp.zeros((B, N, 4), jnp.float32)], axis=-1).reshape(_R, 8)
    wst = W_self.T
    wext = W_edge[:, :C].T
    sm = jnp.zeros((8, C), jnp.float32)
    sm = sm.at[0:3, :].set(W_edge[:, C:C + 3].T)
    sm = sm.at[3, :].set(W_edge[:, C + 3])
    out = _tc_dense(xf, gx, gm, pq, wst, wext, sm)
    return out.reshape(B, N, C)


# speedup vs baseline: 37.4261x; 1.7248x over previous
import functools

import jax
import jax.numpy as jnp
from jax import lax
from jax.experimental import pallas as pl
from jax.experimental.pallas import tpu as pltpu
from jax.experimental.pallas import tpu_sc as plsc

_B, _N, _C, _K = 4, 10000, 128, 16
_DW = 80
_NC, _NS = 2, 16
_NP = 10240
_RP = _B * _NP
_CH = 8
_PER_T = _NP // _NS
_NCHUNK = _PER_T // _CH
_TROWS = _N // _NS
_HIMASK = -65536


@functools.cache
def _make_sc_gather_mean():
    mesh = plsc.VectorSubcoreMesh(
        core_axis_name="c", subcore_axis_name="s",
        num_cores=_NC, num_subcores=_NS)

    @functools.partial(
        pl.kernel,
        out_type=(
            jax.ShapeDtypeStruct((_RP, _C), jnp.float32),
            jax.ShapeDtypeStruct((_RP, 32), jnp.float32),
        ),
        mesh=mesh,
        compiler_params=pltpu.CompilerParams(
            use_tc_tiling_on_sc=False, needs_layout_passes=False),
        scratch_types=[
            pltpu.VMEM((_PER_T * _K,), jnp.int32),
            pltpu.VMEM((2, _CH * _K, _DW), jnp.int32),
            pltpu.VMEM((2, _CH, _C), jnp.float32),
            pltpu.VMEM((2, _CH, 32), jnp.float32),
            pltpu.VMEM_SHARED((_N, _DW), jnp.int32),
            pltpu.SemaphoreType.DMA((2,)),
            pltpu.SemaphoreType.DMA((2,)),
            pltpu.SemaphoreType.DMA((2,)),
            pltpu.SemaphoreType.DMA,
        ],
    )
    def _sc_gather_mean(tab_hbm, idxg_hbm, outx_hbm, outm_hbm,
                        idx_v, rows_v, accx_v, accm_v, tab_sh,
                        rsem, sxsem, smsem, tsem):
        c = lax.axis_index("c")
        s = lax.axis_index("s")

        def _gather(j, sl):
            pltpu.async_copy(
                tab_sh.at[idx_v.at[pl.ds(j * _CH * _K, _CH * _K)]],
                rows_v.at[sl], rsem.at[sl])

        def _gather_wait(sl):
            pltpu.make_async_copy(
                tab_sh.at[idx_v.at[pl.ds(0, _CH * _K)]],
                rows_v.at[sl], rsem.at[sl]).wait()

        inv_k = 1.0 / _K

        for phase in range(2):
            b = c * 2 + phase
            pltpu.async_copy(
                tab_hbm.at[pl.ds(b * _N + s * _TROWS, _TROWS)],
                tab_sh.at[pl.ds(s * _TROWS, _TROWS)], tsem).wait()
            plsc.subcore_barrier()

            base = b * _NP + s * _PER_T
            pltpu.sync_copy(idxg_hbm.at[pl.ds(base * _K, _PER_T * _K)], idx_v)

            def _store(j, sl, base=base):
                row0 = base + j * _CH
                pltpu.async_copy(accx_v.at[sl],
                                 outx_hbm.at[pl.ds(row0, _CH)], sxsem.at[sl])
                pltpu.async_copy(accm_v.at[sl],
                                 outm_hbm.at[pl.ds(row0, _CH)], smsem.at[sl])

            def _store_wait(sl):
                pltpu.make_async_copy(
                    accx_v.at[sl], outx_hbm.at[pl.ds(0, _CH)],
                    sxsem.at[sl]).wait()
                pltpu.make_async_copy(
                    accm_v.at[sl], outm_hbm.at[pl.ds(0, _CH)],
                    smsem.at[sl]).wait()

            _gather(0, 0)

            @pl.loop(0, _NCHUNK, step=2)
            def _pair(j0):
                for sl in (0, 1):
                    j = j0 + sl

                    @pl.when(j + 1 < _NCHUNK)
                    def _():
                        _gather(j + 1, 1 - sl)

                    _gather_wait(sl)

                    @pl.when(j >= 2)
                    def _():
                        _store_wait(sl)

                    @pl.loop(0, _CH)
                    def _center(cc):
                        r0 = cc * _K
                        for d in range(_DW // 16):
                            w0 = rows_v[sl, r0, pl.ds(d * 16, 16)]
                            alo = plsc.bitcast(w0 << 16, jnp.float32)
                            ahi = plsc.bitcast(w0 & _HIMASK, jnp.float32)
                            for k in range(1, _K):
                                wk = rows_v[sl, r0 + k, pl.ds(d * 16, 16)]
                                alo = alo + plsc.bitcast(wk << 16, jnp.float32)
                                ahi = ahi + plsc.bitcast(wk & _HIMASK,
                                                         jnp.float32)
                            alo = alo * inv_k
                            ahi = ahi * inv_k
                            if d < 4:
                                accx_v[sl, cc, pl.ds(d * 16, 16)] = alo
                                accx_v[sl, cc, pl.ds(64 + d * 16, 16)] = ahi
                            else:
                                accm_v[sl, cc, pl.ds(0, 16)] = alo
                                accm_v[sl, cc, pl.ds(16, 16)] = ahi

                    _store(j, sl)

            _store_wait(0)
            _store_wait(1)
            plsc.subcore_barrier()

    return _sc_gather_mean


def _tc_body(x_ref, gx_ref, gm_ref, pq_ref, wst_ref, wext_ref, sm_ref, o_ref):
    xr = x_ref[...]
    acc = jnp.dot(xr, wst_ref[...], preferred_element_type=jnp.float32)
    acc = acc + jnp.dot(gx_ref[...], wext_ref[...],
                        preferred_element_type=jnp.float32)
    gm = gm_ref[...]
    pq = pq_ref[...]
    px, py, pz, q = pq[:, 0:1], pq[:, 1:2], pq[:, 2:3], pq[:, 3:4]
    mpx, mpz = gm[:, 0:1], gm[:, 1:2]
    mpy, mq = gm[:, 16:17], gm[:, 17:18]
    dterm = q - 2.0 * (px * mpx + py * mpy + pz * mpz) + mq
    sm = sm_ref[...]
    acc = acc + ((px - mpx) * sm[0:1, :] + (py - mpy) * sm[1:2, :]
                 + (pz - mpz) * sm[2:3, :] + dterm * sm[3:4, :])
    o_ref[...] = jnp.where(acc >= 0, acc, 0.2 * acc)


_BR = 1024

_tc_dense = pl.pallas_call(
    _tc_body,
    grid=(_RP // _BR,),
    in_specs=[
        pl.BlockSpec((_BR, _C), lambda i: (i, 0)),
        pl.BlockSpec((_BR, _C), lambda i: (i, 0)),
        pl.BlockSpec((_BR, 32), lambda i: (i, 0)),
        pl.BlockSpec((_BR, 8), lambda i: (i, 0)),
        pl.BlockSpec((_C, _C), lambda i: (0, 0)),
        pl.BlockSpec((_C, _C), lambda i: (0, 0)),
        pl.BlockSpec((8, _C), lambda i: (0, 0)),
    ],
    out_specs=pl.BlockSpec((_BR, _C), lambda i: (i, 0)),
    out_shape=jax.ShapeDtypeStruct((_RP, _C), jnp.float32),
)


def kernel(x, pos, idx, W_self, W_edge):
    B, N, C = x.shape
    q = jnp.sum(pos * pos, axis=-1, keepdims=True)
    row_bf = jnp.concatenate(
        [x, pos, q, jnp.zeros((B, N, 2 * _DW - C - 4), jnp.float32)],
        axis=-1).astype(jnp.bfloat16)
    tab = lax.bitcast_convert_type(
        row_bf.reshape(B * N, _DW, 2), jnp.int32)
    idxg = jnp.pad(idx, ((0, 0), (0, _NP - N), (0, 0))).reshape(_RP * _K)
    gx, gm = _make_sc_gather_mean()(tab, idxg)

    xf = jnp.pad(x, ((0, 0), (0, _NP - N), (0, 0))).reshape(_RP, C)
    pq = jnp.pad(jnp.concatenate([pos, q], axis=-1),
                 ((0, 0), (0, _NP - N), (0, 4))).reshape(_RP, 8)
    wst = W_self.T
    perm = jnp.concatenate([jnp.arange(0, C, 2), jnp.arange(1, C, 2)])
    wext = W_edge[:, :C].T[perm, :]
    sm = jnp.zeros((8, C), jnp.float32)
    sm = sm.at[0:3, :].set(W_edge[:, C:C + 3].T)
    sm = sm.at[3, :].set(W_edge[:, C + 3])
    out = _tc_dense(xf, gx, gm, pq, wst, wext, sm)
    return out.reshape(B, _NP, C)[:, :N, :]


# speedup vs baseline: 39.1261x; 1.0454x over previous
import functools

import jax
import jax.numpy as jnp
from jax import lax
from jax.experimental import pallas as pl
from jax.experimental.pallas import tpu as pltpu
from jax.experimental.pallas import tpu_sc as plsc

_B, _N, _C, _K = 4, 10000, 128, 16
_DW = 80
_NC, _NS = 2, 16
_NP = 10240
_RP = _B * _NP
_CH = 8
_PER_T = _NP // _NS
_NCHUNK = _PER_T // _CH
_TROWS = _N // _NS
_HIMASK = -65536


@functools.cache
def _make_sc_gather_mean():
    mesh = plsc.VectorSubcoreMesh(
        core_axis_name="c", subcore_axis_name="s",
        num_cores=_NC, num_subcores=_NS)

    @functools.partial(
        pl.kernel,
        out_type=(
            jax.ShapeDtypeStruct((_RP, _C), jnp.float32),
            jax.ShapeDtypeStruct((_RP, 32), jnp.float32),
        ),
        mesh=mesh,
        compiler_params=pltpu.CompilerParams(
            use_tc_tiling_on_sc=False, needs_layout_passes=False),
        scratch_types=[
            pltpu.VMEM((_PER_T * _K,), jnp.int32),
            pltpu.VMEM((2, _CH * _K, _DW), jnp.int32),
            pltpu.VMEM((2, _CH, _C), jnp.float32),
            pltpu.VMEM((2, _CH, 32), jnp.float32),
            pltpu.VMEM_SHARED((_N, _DW), jnp.int32),
            pltpu.SemaphoreType.DMA((2,)),
            pltpu.SemaphoreType.DMA((2,)),
            pltpu.SemaphoreType.DMA((2,)),
            pltpu.SemaphoreType.DMA,
        ],
    )
    def _sc_gather_mean(tab_hbm, idxg_hbm, outx_hbm, outm_hbm,
                        idx_v, rows_v, accx_v, accm_v, tab_sh,
                        rsem, sxsem, smsem, tsem):
        c = lax.axis_index("c")
        s = lax.axis_index("s")

        def _gather(j, sl):
            pltpu.async_copy(
                tab_sh.at[idx_v.at[pl.ds(j * _CH * _K, _CH * _K)]],
                rows_v.at[sl], rsem.at[sl])

        def _gather_wait(sl):
            pltpu.make_async_copy(
                tab_sh.at[idx_v.at[pl.ds(0, _CH * _K)]],
                rows_v.at[sl], rsem.at[sl]).wait()

        inv_k = 1.0 / _K

        cbase = jnp.minimum(s * _PER_T, _N - _PER_T)

        for phase in range(2):
            b = c * 2 + phase
            pltpu.async_copy(
                tab_hbm.at[pl.ds(b * _N + s * _TROWS, _TROWS)],
                tab_sh.at[pl.ds(s * _TROWS, _TROWS)], tsem).wait()
            plsc.subcore_barrier()

            pltpu.sync_copy(
                idxg_hbm.at[pl.ds((b * _N + cbase) * _K, _PER_T * _K)], idx_v)
            base = b * _NP + cbase

            def _store(j, sl, base=base):
                row0 = base + j * _CH
                pltpu.async_copy(accx_v.at[sl],
                                 outx_hbm.at[pl.ds(row0, _CH)], sxsem.at[sl])
                pltpu.async_copy(accm_v.at[sl],
                                 outm_hbm.at[pl.ds(row0, _CH)], smsem.at[sl])

            def _store_wait(sl):
                pltpu.make_async_copy(
                    accx_v.at[sl], outx_hbm.at[pl.ds(0, _CH)],
                    sxsem.at[sl]).wait()
                pltpu.make_async_copy(
                    accm_v.at[sl], outm_hbm.at[pl.ds(0, _CH)],
                    smsem.at[sl]).wait()

            _gather(0, 0)

            @pl.loop(0, _NCHUNK, step=2)
            def _pair(j0):
                for sl in (0, 1):
                    j = j0 + sl

                    @pl.when(j + 1 < _NCHUNK)
                    def _():
                        _gather(j + 1, 1 - sl)

                    _gather_wait(sl)

                    @pl.when(j >= 2)
                    def _():
                        _store_wait(sl)

                    @pl.loop(0, _CH)
                    def _center(cc):
                        r0 = cc * _K
                        for d in range(_DW // 16):
                            w0 = rows_v[sl, r0, pl.ds(d * 16, 16)]
                            alo = plsc.bitcast(w0 << 16, jnp.float32)
                            ahi = plsc.bitcast(w0 & _HIMASK, jnp.float32)
                            for k in range(1, _K):
                                wk = rows_v[sl, r0 + k, pl.ds(d * 16, 16)]
                                alo = alo + plsc.bitcast(wk << 16, jnp.float32)
                                ahi = ahi + plsc.bitcast(wk & _HIMASK,
                                                         jnp.float32)
                            alo = alo * inv_k
                            ahi = ahi * inv_k
                            if d < 4:
                                accx_v[sl, cc, pl.ds(d * 16, 16)] = alo
                                accx_v[sl, cc, pl.ds(64 + d * 16, 16)] = ahi
                            else:
                                accm_v[sl, cc, pl.ds(0, 16)] = alo
                                accm_v[sl, cc, pl.ds(16, 16)] = ahi

                    _store(j, sl)

            _store_wait(0)
            _store_wait(1)
            plsc.subcore_barrier()

    return _sc_gather_mean


def _tc_body(x_ref, gx_ref, gm_ref, pq_ref, wst_ref, wext_ref, sm_ref, o_ref):
    xr = x_ref[0]
    acc = jnp.dot(xr, wst_ref[...], preferred_element_type=jnp.float32)
    acc = acc + jnp.dot(gx_ref[0], wext_ref[...],
                        preferred_element_type=jnp.float32)
    gm = gm_ref[0]
    pq = pq_ref[0]
    px, py, pz, q = pq[:, 0:1], pq[:, 1:2], pq[:, 2:3], pq[:, 3:4]
    mpx, mpz = gm[:, 0:1], gm[:, 1:2]
    mpy, mq = gm[:, 16:17], gm[:, 17:18]
    dterm = q - 2.0 * (px * mpx + py * mpy + pz * mpz) + mq
    sm = sm_ref[...]
    acc = acc + ((px - mpx) * sm[0:1, :] + (py - mpy) * sm[1:2, :]
                 + (pz - mpz) * sm[2:3, :] + dterm * sm[3:4, :])
    o_ref[0] = jnp.where(acc >= 0, acc, 0.2 * acc)


_BR = 2000

_tc_dense = pl.pallas_call(
    _tc_body,
    grid=(_B, _N // _BR),
    in_specs=[
        pl.BlockSpec((1, _BR, _C), lambda b, i: (b, i, 0)),
        pl.BlockSpec((1, _BR, _C), lambda b, i: (b, i, 0)),
        pl.BlockSpec((1, _BR, 32), lambda b, i: (b, i, 0)),
        pl.BlockSpec((1, _BR, 8), lambda b, i: (b, i, 0)),
        pl.BlockSpec((_C, _C), lambda b, i: (0, 0)),
        pl.BlockSpec((_C, _C), lambda b, i: (0, 0)),
        pl.BlockSpec((8, _C), lambda b, i: (0, 0)),
    ],
    out_specs=pl.BlockSpec((1, _BR, _C), lambda b, i: (b, i, 0)),
    out_shape=jax.ShapeDtypeStruct((_B, _N, _C), jnp.float32),
)


def kernel(x, pos, idx, W_self, W_edge):
    B, N, C = x.shape
    q = jnp.sum(pos * pos, axis=-1, keepdims=True)
    row_bf = jnp.concatenate(
        [x, pos, q, jnp.zeros((B, N, 2 * _DW - C - 4), jnp.float32)],
        axis=-1).astype(jnp.bfloat16)
    tab = lax.bitcast_convert_type(
        row_bf.reshape(B * N, _DW, 2), jnp.int32)
    idxg = idx.reshape(B * N * _K)
    gx, gm = _make_sc_gather_mean()(tab, idxg)
    gx = gx.reshape(B, _NP, C)
    gm = gm.reshape(B, _NP, 32)

    pq = jnp.concatenate(
        [pos, q, jnp.zeros((B, N, 4), jnp.float32)], axis=-1)
    wst = W_self.T
    perm = jnp.concatenate([jnp.arange(0, C, 2), jnp.arange(1, C, 2)])
    wext = W_edge[:, :C].T[perm, :]
    sm = jnp.zeros((8, C), jnp.float32)
    sm = sm.at[0:3, :].set(W_edge[:, C:C + 3].T)
    sm = sm.at[3, :].set(W_edge[:, C + 3])
    return _tc_dense(x, gx, gm, pq, wst, wext, sm)
